# baseline (speedup 1.0000x reference)
import functools

import jax
import jax.numpy as jnp
from jax.experimental import pallas as pl
from jax.experimental.pallas import tpu as pltpu

_K = 1024
_D = 32
_TILE = 512


def _vq_kernel(x_ref, a_ref, b_ref, w_ref, out_ref, loss_ref,
               counts_ref, sq_ref):
    i = pl.program_id(0)
    nsteps = pl.num_programs(0)

    @pl.when(i == 0)
    def _init():
        counts_ref[...] = jnp.zeros_like(counts_ref)
        sq_ref[...] = jnp.zeros_like(sq_ref)

    x_t = x_ref[...]
    w = w_ref[...]
    c = jax.lax.dot_general(x_t, w, dimension_numbers=(((1,), (1,)), ((), ())))
    d = (a_ref[...] + b_ref[...]) - 2.0 * c
    m = jnp.min(d, axis=1, keepdims=True)
    iota = jax.lax.broadcasted_iota(jnp.int32, d.shape, 1)
    sel = jnp.where(d == m, iota, _K)
    amin = jnp.min(sel, axis=1, keepdims=True)
    onehot = (iota == amin).astype(jnp.float32)
    q = jax.lax.dot_general(onehot, w, dimension_numbers=(((1,), (0,)), ((), ())))
    diff = q - x_t
    out_ref[...] = x_t + diff
    counts_ref[...] = counts_ref[...] + jnp.sum(onehot, axis=0, keepdims=True)
    sq_ref[...] = sq_ref[...] + jnp.sum(diff * diff, axis=0, keepdims=True)

    @pl.when(i == nsteps - 1)
    def _finalize():
        n_total = nsteps * _TILE
        p = counts_ref[...] * (1.0 / n_total)
        mse = jnp.sum(sq_ref[...]) / (n_total * _D)
        loss = mse + 0.25 * mse
        entropy = -jnp.sum(p * jnp.log(p + 1e-10))
        div = jnp.sum((p - 1.0 / _K) ** 2)
        kl = jnp.sum(p * jnp.log(p * float(_K) + 1e-10))
        loss_ref[0, 0] = ((loss - entropy) + div) + kl


@functools.partial(jax.jit)
def kernel(x, W):
    flat_x = x.reshape(-1, _D)
    n = flat_x.shape[0]
    a = jnp.sum(flat_x ** 2, axis=1, keepdims=True)
    b = jnp.sum(W ** 2, axis=1)[None, :]
    out_q, out_loss = pl.pallas_call(
        _vq_kernel,
        grid=(n // _TILE,),
        in_specs=[
            pl.BlockSpec((_TILE, _D), lambda i: (i, 0)),
            pl.BlockSpec((_TILE, 1), lambda i: (i, 0)),
            pl.BlockSpec((1, _K), lambda i: (0, 0)),
            pl.BlockSpec((_K, _D), lambda i: (0, 0)),
        ],
        out_specs=[
            pl.BlockSpec((_TILE, _D), lambda i: (i, 0)),
            pl.BlockSpec(memory_space=pltpu.SMEM),
        ],
        out_shape=[
            jax.ShapeDtypeStruct((n, _D), jnp.float32),
            jax.ShapeDtypeStruct((1, 1), jnp.float32),
        ],
        scratch_shapes=[
            pltpu.VMEM((1, _K), jnp.float32),
            pltpu.VMEM((1, _D), jnp.float32),
        ],
    )(flat_x, a, b, W)
    return out_q.reshape(x.shape), out_loss.reshape(())


# speedup vs baseline: 1.1477x; 1.1477x over previous
import functools

import jax
import jax.numpy as jnp
from jax.experimental import pallas as pl
from jax.experimental.pallas import tpu as pltpu

_K = 1024
_D = 32
_TILE = 512


def _vq_kernel(x_ref, a_ref, b_ref, iota_ref, w_ref, out_ref, loss_ref,
               counts_ref, sq_ref):
    i = pl.program_id(0)
    nsteps = pl.num_programs(0)

    @pl.when(i == 0)
    def _init():
        counts_ref[...] = jnp.zeros_like(counts_ref)
        sq_ref[...] = jnp.zeros_like(sq_ref)

    x_t = x_ref[...]
    w = w_ref[...]
    c = jax.lax.dot_general(x_t, w, dimension_numbers=(((1,), (1,)), ((), ())))
    d = (a_ref[...] + b_ref[...]) - 2.0 * c
    m = jnp.min(d, axis=1, keepdims=True)
    iota = iota_ref[...]
    sel = jnp.where(d == m, iota, float(_K))
    amin = jnp.min(sel, axis=1, keepdims=True)
    onehot = (iota == amin).astype(jnp.float32)
    q = jax.lax.dot_general(onehot, w, dimension_numbers=(((1,), (0,)), ((), ())))
    diff = q - x_t
    out_ref[...] = x_t + diff
    ones_row = jnp.ones((1, x_t.shape[0]), jnp.float32)
    csum = jax.lax.dot_general(ones_row, onehot,
                               dimension_numbers=(((1,), (0,)), ((), ())))
    counts_ref[...] = counts_ref[...] + csum
    sq_ref[...] = sq_ref[...] + jnp.sum(diff * diff, axis=0, keepdims=True)

    @pl.when(i == nsteps - 1)
    def _finalize():
        n_total = nsteps * _TILE
        p = counts_ref[...] * (1.0 / n_total)
        mse = jnp.sum(sq_ref[...]) / (n_total * _D)
        loss = mse + 0.25 * mse
        entropy = -jnp.sum(p * jnp.log(p + 1e-10))
        div = jnp.sum((p - 1.0 / _K) ** 2)
        kl = jnp.sum(p * jnp.log(p * float(_K) + 1e-10))
        loss_ref[0, 0] = ((loss - entropy) + div) + kl


@functools.partial(jax.jit)
def kernel(x, W):
    flat_x = x.reshape(-1, _D)
    n = flat_x.shape[0]
    a = jnp.sum(flat_x ** 2, axis=1, keepdims=True)
    b = jnp.sum(W ** 2, axis=1)[None, :]
    iota = jnp.arange(_K, dtype=jnp.float32)[None, :]
    out_q, out_loss = pl.pallas_call(
        _vq_kernel,
        grid=(n // _TILE,),
        in_specs=[
            pl.BlockSpec((_TILE, _D), lambda i: (i, 0)),
            pl.BlockSpec((_TILE, 1), lambda i: (i, 0)),
            pl.BlockSpec((1, _K), lambda i: (0, 0)),
            pl.BlockSpec((1, _K), lambda i: (0, 0)),
            pl.BlockSpec((_K, _D), lambda i: (0, 0)),
        ],
        out_specs=[
            pl.BlockSpec((_TILE, _D), lambda i: (i, 0)),
            pl.BlockSpec(memory_space=pltpu.SMEM),
        ],
        out_shape=[
            jax.ShapeDtypeStruct((n, _D), jnp.float32),
            jax.ShapeDtypeStruct((1, 1), jnp.float32),
        ],
        scratch_shapes=[
            pltpu.VMEM((1, _K), jnp.float32),
            pltpu.VMEM((1, _D), jnp.float32),
        ],
    )(flat_x, a, b, iota, W)
    return out_q.reshape(x.shape), out_loss.reshape(())
